# baseline (speedup 1.0000x reference)
import functools

import jax
import jax.numpy as jnp
from jax import lax
from jax.experimental import pallas as pl
from jax.experimental.pallas import tpu as pltpu
from jax.experimental.pallas import tpu_sc as plsc

B = 16384
EMB = 64
HID = 128
NV = 1000000

_IDX_CHUNK = 128

_TW = 8192
_TGRID = -(-NV // _TW)
_QB = _TW // 4
NQ = _TGRID * _QB


def _transpack_body(u_ref, i_ref, qu_ref, qi_ref):
    def rne(v):
        b = lax.bitcast_convert_type(v, jnp.uint32)
        return (b + jnp.uint32(0x7FFF) + ((b >> 16) & jnp.uint32(1))) \
            & jnp.uint32(0xFFFF0000)

    for src, dst in ((u_ref, qu_ref), (i_ref, qi_ref)):
        x = src[...]
        hi = rne(x[: EMB // 2, :])
        lo = rne(x[EMB // 2:, :])
        w = hi | (lo >> 16)
        wf = lax.bitcast_convert_type(w, jnp.float32)
        wb = jnp.concatenate(
            [wf[:, s * _QB:(s + 1) * _QB] for s in range(4)], axis=0)
        dst[...] = wb.T


def _transpack(ut, it):
    return pl.pallas_call(
        _transpack_body,
        grid=(_TGRID,),
        in_specs=[
            pl.BlockSpec((EMB, _TW), lambda j: (0, j)),
            pl.BlockSpec((EMB, _TW), lambda j: (0, j)),
        ],
        out_specs=[
            pl.BlockSpec((_QB, 128), lambda j: (j, 0)),
            pl.BlockSpec((_QB, 128), lambda j: (j, 0)),
        ],
        out_shape=[
            jax.ShapeDtypeStruct((NQ, 128), jnp.float32),
            jax.ShapeDtypeStruct((NQ, 128), jnp.float32),
        ],
    )(ut, it)


def _make_gather():
    nc, ns = 2, 16
    nw = nc * ns
    b_per_w = B // nw
    n_chunks = b_per_w // _IDX_CHUNK

    mesh = plsc.VectorSubcoreMesh(core_axis_name="c", subcore_axis_name="s")

    @functools.partial(
        pl.kernel,
        out_type=[
            jax.ShapeDtypeStruct((B, 128), jnp.float32),
            jax.ShapeDtypeStruct((B, 128), jnp.float32),
        ],
        mesh=mesh,
        scratch_types=[
            pltpu.VMEM((n_chunks, _IDX_CHUNK), jnp.int32),
            pltpu.VMEM((n_chunks, _IDX_CHUNK), jnp.int32),
            pltpu.VMEM((2, _IDX_CHUNK, 128), jnp.float32),
            pltpu.VMEM((2, _IDX_CHUNK, 128), jnp.float32),
            pltpu.SemaphoreType.DMA,
        ],
    )
    def gather_kernel(qu_hbm, qi_hbm, uid_hbm, iid_hbm, out_u, out_i,
                      uidx_v, iidx_v, urows_v, irows_v, sem):
        wid = lax.axis_index("s") * nc + lax.axis_index("c")
        rbase = wid * n_chunks
        base = wid * b_per_w

        pltpu.sync_copy(uid_hbm.at[pl.ds(rbase, n_chunks)], uidx_v)
        pltpu.sync_copy(iid_hbm.at[pl.ds(rbase, n_chunks)], iidx_v)

        for p in range(n_chunks // 2):
            copies = []
            for k in range(2):
                j = 2 * p + k
                copies.append(pltpu.async_copy(
                    qu_hbm.at[uidx_v.at[j]], urows_v.at[k], sem))
                copies.append(pltpu.async_copy(
                    qi_hbm.at[iidx_v.at[j]], irows_v.at[k], sem))
            for c in copies:
                c.wait()
            off = base + p * 2 * _IDX_CHUNK
            for k in range(2):
                pltpu.sync_copy(
                    urows_v.at[k],
                    out_u.at[pl.ds(off + k * _IDX_CHUNK, _IDX_CHUNK)])
                pltpu.sync_copy(
                    irows_v.at[k],
                    out_i.at[pl.ds(off + k * _IDX_CHUNK, _IDX_CHUNK)])

    return gather_kernel


def _unpack_half(q_ref, m0_ref, m1_ref):
    q = q_ref[...]
    m0 = m0_ref[...] != 0
    m1 = m1_ref[...] != 0
    ab = jnp.where(m0, q[:, 32:64], q[:, 0:32])
    cd = jnp.where(m0, q[:, 96:128], q[:, 64:96])
    bits = lax.bitcast_convert_type(
        jnp.where(m1, cd, ab), jnp.uint32)
    hi = lax.bitcast_convert_type(bits & jnp.uint32(0xFFFF0000), jnp.float32)
    lo = lax.bitcast_convert_type(bits << 16, jnp.float32)
    return hi, lo


def _mlp_body(uq_ref, iq_ref, um0_ref, um1_ref, im0_ref, im1_ref,
              w1a_ref, w1b_ref, w1c_ref, w1d_ref, b1_ref, w2_ref, b2_ref,
              w3_ref, b3_ref, o_ref):
    dot = functools.partial(jnp.dot, preferred_element_type=jnp.float32)
    uhi, ulo = _unpack_half(uq_ref, um0_ref, um1_ref)
    ihi, ilo = _unpack_half(iq_ref, im0_ref, im1_ref)
    h = dot(uhi, w1a_ref[...]) + dot(ulo, w1b_ref[...])
    h = h + dot(ihi, w1c_ref[...]) + dot(ilo, w1d_ref[...])
    h = jnp.maximum(h + b1_ref[...], 0.0)
    h2 = jnp.maximum(dot(h, w2_ref[...]) + b2_ref[...], 0.0)
    z = jnp.sum(h2 * w3_ref[...], axis=1) + b3_ref[0, 0]
    o_ref[...] = 1.0 / (1.0 + jnp.exp(-z))


def _mlp(uq, iq, um0, um1, im0, im1, w1a, w1b, w1c, w1d, b1, w2, b2, w3, b3):
    blk = 4096
    grid = (B // blk,)
    return pl.pallas_call(
        _mlp_body,
        grid=grid,
        in_specs=[
            pl.BlockSpec((blk, 128), lambda k: (k, 0)),
            pl.BlockSpec((blk, 128), lambda k: (k, 0)),
            pl.BlockSpec((blk, 1), lambda k: (k, 0)),
            pl.BlockSpec((blk, 1), lambda k: (k, 0)),
            pl.BlockSpec((blk, 1), lambda k: (k, 0)),
            pl.BlockSpec((blk, 1), lambda k: (k, 0)),
            pl.BlockSpec((EMB // 2, HID), lambda k: (0, 0)),
            pl.BlockSpec((EMB // 2, HID), lambda k: (0, 0)),
            pl.BlockSpec((EMB // 2, HID), lambda k: (0, 0)),
            pl.BlockSpec((EMB // 2, HID), lambda k: (0, 0)),
            pl.BlockSpec((1, HID), lambda k: (0, 0)),
            pl.BlockSpec((HID, HID // 2), lambda k: (0, 0)),
            pl.BlockSpec((1, HID // 2), lambda k: (0, 0)),
            pl.BlockSpec((1, HID // 2), lambda k: (0, 0)),
            pl.BlockSpec((1, 1), lambda k: (0, 0)),
        ],
        out_specs=pl.BlockSpec((blk,), lambda k: (k,)),
        out_shape=jax.ShapeDtypeStruct((B,), jnp.float32),
    )(uq, iq, um0, um1, im0, im1, w1a, w1b, w1c, w1d, b1, w2, b2, w3, b3)


def kernel(user_ids, item_ids, user_emb, item_emb, W1, b1, W2, b2, W3, b3):
    uq_row = (user_ids // _TW) * _QB + (user_ids % _QB)
    iq_row = (item_ids // _TW) * _QB + (item_ids % _QB)
    uidq = uq_row.reshape(B // _IDX_CHUNK, _IDX_CHUNK)
    iidq = iq_row.reshape(B // _IDX_CHUNK, _IDX_CHUNK)
    qu, qi = _transpack(user_emb.T, item_emb.T)
    uq, iq = _make_gather()(qu, qi, uidq, iidq)
    su = (user_ids % _TW) // _QB
    si = (item_ids % _TW) // _QB
    um0 = (-(su & 1)).reshape(B, 1)
    um1 = (-(su >> 1)).reshape(B, 1)
    im0 = (-(si & 1)).reshape(B, 1)
    im1 = (-(si >> 1)).reshape(B, 1)
    return _mlp(uq, iq, um0, um1, im0, im1,
                W1[0:32], W1[32:64], W1[64:96], W1[96:128],
                b1.reshape(1, HID), W2, b2.reshape(1, HID // 2),
                W3.reshape(1, HID // 2), b3.reshape(1, 1))


# speedup vs baseline: 1.0252x; 1.0252x over previous
import functools

import jax
import jax.numpy as jnp
from jax import lax
from jax.experimental import pallas as pl
from jax.experimental.pallas import tpu as pltpu
from jax.experimental.pallas import tpu_sc as plsc

B = 16384
EMB = 64
HID = 128
NV = 1000000

_IDX_CHUNK = 128

_TW = 8192
_TGRID = -(-NV // _TW)
_QB = _TW // 4
NQ = _TGRID * _QB


def _transpack_body(u_ref, i_ref, qu_ref, qi_ref):
    def rne(v):
        b = lax.bitcast_convert_type(v, jnp.uint32)
        return (b + jnp.uint32(0x7FFF) + ((b >> 16) & jnp.uint32(1))) \
            & jnp.uint32(0xFFFF0000)

    for src, dst in ((u_ref, qu_ref), (i_ref, qi_ref)):
        x = src[...]
        hi = rne(x[: EMB // 2, :])
        lo = rne(x[EMB // 2:, :])
        w = hi | (lo >> 16)
        wf = lax.bitcast_convert_type(w, jnp.float32)
        wb = jnp.concatenate(
            [wf[:, s * _QB:(s + 1) * _QB] for s in range(4)], axis=0)
        dst[...] = wb.T


def _transpack(ut, it):
    return pl.pallas_call(
        _transpack_body,
        grid=(_TGRID,),
        in_specs=[
            pl.BlockSpec((EMB, _TW), lambda j: (0, j)),
            pl.BlockSpec((EMB, _TW), lambda j: (0, j)),
        ],
        out_specs=[
            pl.BlockSpec((_QB, 128), lambda j: (j, 0)),
            pl.BlockSpec((_QB, 128), lambda j: (j, 0)),
        ],
        out_shape=[
            jax.ShapeDtypeStruct((NQ, 128), jnp.float32),
            jax.ShapeDtypeStruct((NQ, 128), jnp.float32),
        ],
    )(ut, it)


def _make_gather():
    nc, ns = 2, 16
    nw = nc * ns
    b_per_w = B // nw
    n_chunks = b_per_w // _IDX_CHUNK

    mesh = plsc.VectorSubcoreMesh(core_axis_name="c", subcore_axis_name="s")

    @functools.partial(
        pl.kernel,
        out_type=[
            jax.ShapeDtypeStruct((B, 128), jnp.float32),
            jax.ShapeDtypeStruct((B, 128), jnp.float32),
        ],
        mesh=mesh,
        scratch_types=[
            pltpu.VMEM((n_chunks, _IDX_CHUNK), jnp.int32),
            pltpu.VMEM((n_chunks, _IDX_CHUNK), jnp.int32),
            pltpu.VMEM((2, _IDX_CHUNK, 128), jnp.float32),
            pltpu.VMEM((2, _IDX_CHUNK, 128), jnp.float32),
            pltpu.SemaphoreType.DMA,
        ],
    )
    def gather_kernel(qu_hbm, qi_hbm, uid_hbm, iid_hbm, out_u, out_i,
                      uidx_v, iidx_v, urows_v, irows_v, sem):
        wid = lax.axis_index("s") * nc + lax.axis_index("c")
        rbase = wid * n_chunks
        base = wid * b_per_w

        pltpu.sync_copy(uid_hbm.at[pl.ds(rbase, n_chunks)], uidx_v)
        pltpu.sync_copy(iid_hbm.at[pl.ds(rbase, n_chunks)], iidx_v)

        for p in range(n_chunks // 2):
            copies = []
            for k in range(2):
                j = 2 * p + k
                copies.append(pltpu.async_copy(
                    qu_hbm.at[uidx_v.at[j]], urows_v.at[k], sem))
                copies.append(pltpu.async_copy(
                    qi_hbm.at[iidx_v.at[j]], irows_v.at[k], sem))
            for c in copies:
                c.wait()
            off = base + p * 2 * _IDX_CHUNK
            for k in range(2):
                pltpu.sync_copy(
                    urows_v.at[k],
                    out_u.at[pl.ds(off + k * _IDX_CHUNK, _IDX_CHUNK)])
                pltpu.sync_copy(
                    irows_v.at[k],
                    out_i.at[pl.ds(off + k * _IDX_CHUNK, _IDX_CHUNK)])

    return gather_kernel


def _unpack_half(q_ref, s_ref):
    q = q_ref[...]
    s = s_ref[...]
    ab = jnp.where(s == 0, q[:, 0:32], q[:, 32:64])
    cd = jnp.where(s == 2, q[:, 64:96], q[:, 96:128])
    bits = lax.bitcast_convert_type(
        jnp.where(s < 2, ab, cd), jnp.uint32)
    hi = lax.bitcast_convert_type(bits & jnp.uint32(0xFFFF0000), jnp.float32)
    lo = lax.bitcast_convert_type(bits << 16, jnp.float32)
    return hi, lo


def _mlp_body(uq_ref, iq_ref, su_ref, si_ref,
              w1a_ref, w1b_ref, w1c_ref, w1d_ref, b1_ref, w2_ref, b2_ref,
              w3_ref, b3_ref, o_ref):
    dot = functools.partial(jnp.dot, preferred_element_type=jnp.float32)
    uhi, ulo = _unpack_half(uq_ref, su_ref)
    ihi, ilo = _unpack_half(iq_ref, si_ref)
    h = dot(uhi, w1a_ref[...]) + dot(ulo, w1b_ref[...])
    h = h + dot(ihi, w1c_ref[...]) + dot(ilo, w1d_ref[...])
    h = jnp.maximum(h + b1_ref[...], 0.0)
    h2 = jnp.maximum(dot(h, w2_ref[...]) + b2_ref[...], 0.0)
    z = jnp.sum(h2 * w3_ref[...], axis=1) + b3_ref[0, 0]
    o_ref[...] = 1.0 / (1.0 + jnp.exp(-z))


def _mlp(uq, iq, su, si, w1a, w1b, w1c, w1d, b1, w2, b2, w3, b3):
    blk = 4096
    grid = (B // blk,)
    return pl.pallas_call(
        _mlp_body,
        grid=grid,
        in_specs=[
            pl.BlockSpec((blk, 128), lambda k: (k, 0)),
            pl.BlockSpec((blk, 128), lambda k: (k, 0)),
            pl.BlockSpec((blk, 1), lambda k: (k, 0)),
            pl.BlockSpec((blk, 1), lambda k: (k, 0)),
            pl.BlockSpec((EMB // 2, HID), lambda k: (0, 0)),
            pl.BlockSpec((EMB // 2, HID), lambda k: (0, 0)),
            pl.BlockSpec((EMB // 2, HID), lambda k: (0, 0)),
            pl.BlockSpec((EMB // 2, HID), lambda k: (0, 0)),
            pl.BlockSpec((1, HID), lambda k: (0, 0)),
            pl.BlockSpec((HID, HID // 2), lambda k: (0, 0)),
            pl.BlockSpec((1, HID // 2), lambda k: (0, 0)),
            pl.BlockSpec((1, HID // 2), lambda k: (0, 0)),
            pl.BlockSpec((1, 1), lambda k: (0, 0)),
        ],
        out_specs=pl.BlockSpec((blk,), lambda k: (k,)),
        out_shape=jax.ShapeDtypeStruct((B,), jnp.float32),
    )(uq, iq, su, si, w1a, w1b, w1c, w1d, b1, w2, b2, w3, b3)


def kernel(user_ids, item_ids, user_emb, item_emb, W1, b1, W2, b2, W3, b3):
    uq_row = (user_ids // _TW) * _QB + (user_ids % _QB)
    iq_row = (item_ids // _TW) * _QB + (item_ids % _QB)
    uidq = uq_row.reshape(B // _IDX_CHUNK, _IDX_CHUNK)
    iidq = iq_row.reshape(B // _IDX_CHUNK, _IDX_CHUNK)
    qu, qi = _transpack(user_emb.T, item_emb.T)
    uq, iq = _make_gather()(qu, qi, uidq, iidq)
    su = ((user_ids % _TW) // _QB).reshape(B, 1)
    si = ((item_ids % _TW) // _QB).reshape(B, 1)
    return _mlp(uq, iq, su, si,
                W1[0:32], W1[32:64], W1[64:96], W1[96:128],
                b1.reshape(1, HID), W2, b2.reshape(1, HID // 2),
                W3.reshape(1, HID // 2), b3.reshape(1, 1))


# speedup vs baseline: 1.0669x; 1.0407x over previous
import functools

import jax
import jax.numpy as jnp
from jax import lax
from jax.experimental import pallas as pl
from jax.experimental.pallas import tpu as pltpu
from jax.experimental.pallas import tpu_sc as plsc

B = 16384
EMB = 64
HID = 128
NV = 1000000

_IDX_CHUNK = 128

_TW = 16384
_TGRID = -(-NV // _TW)
_QB = _TW // 4
NQ = _TGRID * _QB


def _transpack_body(u_ref, i_ref, qu_ref, qi_ref):
    def rne(v):
        b = lax.bitcast_convert_type(v, jnp.uint32)
        return (b + jnp.uint32(0x7FFF) + ((b >> 16) & jnp.uint32(1))) \
            & jnp.uint32(0xFFFF0000)

    for src, dst in ((u_ref, qu_ref), (i_ref, qi_ref)):
        x = src[...]
        hi = rne(x[: EMB // 2, :])
        lo = rne(x[EMB // 2:, :])
        w = hi | (lo >> 16)
        wf = lax.bitcast_convert_type(w, jnp.float32)
        wb = jnp.concatenate(
            [wf[:, s * _QB:(s + 1) * _QB] for s in range(4)], axis=0)
        dst[...] = wb.T


def _transpack(ut, it):
    return pl.pallas_call(
        _transpack_body,
        grid=(_TGRID,),
        in_specs=[
            pl.BlockSpec((EMB, _TW), lambda j: (0, j)),
            pl.BlockSpec((EMB, _TW), lambda j: (0, j)),
        ],
        out_specs=[
            pl.BlockSpec((_QB, 128), lambda j: (j, 0)),
            pl.BlockSpec((_QB, 128), lambda j: (j, 0)),
        ],
        out_shape=[
            jax.ShapeDtypeStruct((NQ, 128), jnp.float32),
            jax.ShapeDtypeStruct((NQ, 128), jnp.float32),
        ],
    )(ut, it)


def _make_gather():
    nc, ns = 2, 16
    nw = nc * ns
    b_per_w = B // nw
    n_chunks = b_per_w // _IDX_CHUNK

    mesh = plsc.VectorSubcoreMesh(core_axis_name="c", subcore_axis_name="s")

    @functools.partial(
        pl.kernel,
        out_type=[
            jax.ShapeDtypeStruct((B, 128), jnp.float32),
            jax.ShapeDtypeStruct((B, 128), jnp.float32),
        ],
        mesh=mesh,
        scratch_types=[
            pltpu.VMEM((n_chunks, _IDX_CHUNK), jnp.int32),
            pltpu.VMEM((n_chunks, _IDX_CHUNK), jnp.int32),
            pltpu.VMEM((2, _IDX_CHUNK, 128), jnp.float32),
            pltpu.VMEM((2, _IDX_CHUNK, 128), jnp.float32),
            pltpu.SemaphoreType.DMA,
        ],
    )
    def gather_kernel(qu_hbm, qi_hbm, uid_hbm, iid_hbm, out_u, out_i,
                      uidx_v, iidx_v, urows_v, irows_v, sem):
        wid = lax.axis_index("s") * nc + lax.axis_index("c")
        rbase = wid * n_chunks
        base = wid * b_per_w

        pltpu.sync_copy(uid_hbm.at[pl.ds(rbase, n_chunks)], uidx_v)
        pltpu.sync_copy(iid_hbm.at[pl.ds(rbase, n_chunks)], iidx_v)

        for p in range(n_chunks // 2):
            copies = []
            for k in range(2):
                j = 2 * p + k
                copies.append(pltpu.async_copy(
                    qu_hbm.at[uidx_v.at[j]], urows_v.at[k], sem))
                copies.append(pltpu.async_copy(
                    qi_hbm.at[iidx_v.at[j]], irows_v.at[k], sem))
            for c in copies:
                c.wait()
            off = base + p * 2 * _IDX_CHUNK
            for k in range(2):
                pltpu.sync_copy(
                    urows_v.at[k],
                    out_u.at[pl.ds(off + k * _IDX_CHUNK, _IDX_CHUNK)])
                pltpu.sync_copy(
                    irows_v.at[k],
                    out_i.at[pl.ds(off + k * _IDX_CHUNK, _IDX_CHUNK)])

    return gather_kernel


def _unpack_half(q_ref, s_ref):
    q = q_ref[...]
    s = s_ref[...]
    ab = jnp.where(s == 0, q[:, 0:32], q[:, 32:64])
    cd = jnp.where(s == 2, q[:, 64:96], q[:, 96:128])
    bits = lax.bitcast_convert_type(
        jnp.where(s < 2, ab, cd), jnp.uint32)
    hi = lax.bitcast_convert_type(bits & jnp.uint32(0xFFFF0000), jnp.float32)
    lo = lax.bitcast_convert_type(bits << 16, jnp.float32)
    return hi, lo


def _mlp_body(uq_ref, iq_ref, su_ref, si_ref,
              w1a_ref, w1b_ref, w1c_ref, w1d_ref, b1_ref, w2_ref, b2_ref,
              w3_ref, b3_ref, o_ref):
    dot = functools.partial(jnp.dot, preferred_element_type=jnp.float32)
    uhi, ulo = _unpack_half(uq_ref, su_ref)
    ihi, ilo = _unpack_half(iq_ref, si_ref)
    h = dot(uhi, w1a_ref[...]) + dot(ulo, w1b_ref[...])
    h = h + dot(ihi, w1c_ref[...]) + dot(ilo, w1d_ref[...])
    h = jnp.maximum(h + b1_ref[...], 0.0)
    h2 = jnp.maximum(dot(h, w2_ref[...]) + b2_ref[...], 0.0)
    z = jnp.sum(h2 * w3_ref[...], axis=1) + b3_ref[0, 0]
    o_ref[...] = 1.0 / (1.0 + jnp.exp(-z))


def _mlp(uq, iq, su, si, w1a, w1b, w1c, w1d, b1, w2, b2, w3, b3):
    blk = 8192
    grid = (B // blk,)
    return pl.pallas_call(
        _mlp_body,
        grid=grid,
        in_specs=[
            pl.BlockSpec((blk, 128), lambda k: (k, 0)),
            pl.BlockSpec((blk, 128), lambda k: (k, 0)),
            pl.BlockSpec((blk, 1), lambda k: (k, 0)),
            pl.BlockSpec((blk, 1), lambda k: (k, 0)),
            pl.BlockSpec((EMB // 2, HID), lambda k: (0, 0)),
            pl.BlockSpec((EMB // 2, HID), lambda k: (0, 0)),
            pl.BlockSpec((EMB // 2, HID), lambda k: (0, 0)),
            pl.BlockSpec((EMB // 2, HID), lambda k: (0, 0)),
            pl.BlockSpec((1, HID), lambda k: (0, 0)),
            pl.BlockSpec((HID, HID // 2), lambda k: (0, 0)),
            pl.BlockSpec((1, HID // 2), lambda k: (0, 0)),
            pl.BlockSpec((1, HID // 2), lambda k: (0, 0)),
            pl.BlockSpec((1, 1), lambda k: (0, 0)),
        ],
        out_specs=pl.BlockSpec((blk,), lambda k: (k,)),
        out_shape=jax.ShapeDtypeStruct((B,), jnp.float32),
    )(uq, iq, su, si, w1a, w1b, w1c, w1d, b1, w2, b2, w3, b3)


def kernel(user_ids, item_ids, user_emb, item_emb, W1, b1, W2, b2, W3, b3):
    uq_row = (user_ids // _TW) * _QB + (user_ids % _QB)
    iq_row = (item_ids // _TW) * _QB + (item_ids % _QB)
    uidq = uq_row.reshape(B // _IDX_CHUNK, _IDX_CHUNK)
    iidq = iq_row.reshape(B // _IDX_CHUNK, _IDX_CHUNK)
    qu, qi = _transpack(user_emb.T, item_emb.T)
    uq, iq = _make_gather()(qu, qi, uidq, iidq)
    su = ((user_ids % _TW) // _QB).reshape(B, 1)
    si = ((item_ids % _TW) // _QB).reshape(B, 1)
    return _mlp(uq, iq, su, si,
                W1[0:32], W1[32:64], W1[64:96], W1[96:128],
                b1.reshape(1, HID), W2, b2.reshape(1, HID // 2),
                W3.reshape(1, HID // 2), b3.reshape(1, 1))


# speedup vs baseline: 1.0723x; 1.0051x over previous
import functools

import jax
import jax.numpy as jnp
from jax import lax
from jax.experimental import pallas as pl
from jax.experimental.pallas import tpu as pltpu
from jax.experimental.pallas import tpu_sc as plsc

B = 16384
EMB = 64
HID = 128
NV = 1000000

_IDX_CHUNK = 128

_TW = 32768
_TGRID = -(-NV // _TW)
_QB = _TW // 4
NQ = _TGRID * _QB


def _transpack_body(u_ref, i_ref, qu_ref, qi_ref):
    def rne(v):
        b = lax.bitcast_convert_type(v, jnp.uint32)
        return (b + jnp.uint32(0x7FFF) + ((b >> 16) & jnp.uint32(1))) \
            & jnp.uint32(0xFFFF0000)

    for src, dst in ((u_ref, qu_ref), (i_ref, qi_ref)):
        x = src[...]
        hi = rne(x[: EMB // 2, :])
        lo = rne(x[EMB // 2:, :])
        w = hi | (lo >> 16)
        wf = lax.bitcast_convert_type(w, jnp.float32)
        wb = jnp.concatenate(
            [wf[:, s * _QB:(s + 1) * _QB] for s in range(4)], axis=0)
        dst[...] = wb.T


def _transpack(ut, it):
    return pl.pallas_call(
        _transpack_body,
        grid=(_TGRID,),
        in_specs=[
            pl.BlockSpec((EMB, _TW), lambda j: (0, j)),
            pl.BlockSpec((EMB, _TW), lambda j: (0, j)),
        ],
        out_specs=[
            pl.BlockSpec((_QB, 128), lambda j: (j, 0)),
            pl.BlockSpec((_QB, 128), lambda j: (j, 0)),
        ],
        out_shape=[
            jax.ShapeDtypeStruct((NQ, 128), jnp.float32),
            jax.ShapeDtypeStruct((NQ, 128), jnp.float32),
        ],
    )(ut, it)


def _make_gather():
    nc, ns = 2, 16
    nw = nc * ns
    b_per_w = B // nw
    n_chunks = b_per_w // _IDX_CHUNK

    mesh = plsc.VectorSubcoreMesh(core_axis_name="c", subcore_axis_name="s")

    @functools.partial(
        pl.kernel,
        out_type=[
            jax.ShapeDtypeStruct((B, 128), jnp.float32),
            jax.ShapeDtypeStruct((B, 128), jnp.float32),
        ],
        mesh=mesh,
        scratch_types=[
            pltpu.VMEM((n_chunks, _IDX_CHUNK), jnp.int32),
            pltpu.VMEM((n_chunks, _IDX_CHUNK), jnp.int32),
            pltpu.VMEM((2, _IDX_CHUNK, 128), jnp.float32),
            pltpu.VMEM((2, _IDX_CHUNK, 128), jnp.float32),
            pltpu.SemaphoreType.DMA,
        ],
    )
    def gather_kernel(qu_hbm, qi_hbm, uid_hbm, iid_hbm, out_u, out_i,
                      uidx_v, iidx_v, urows_v, irows_v, sem):
        wid = lax.axis_index("s") * nc + lax.axis_index("c")
        rbase = wid * n_chunks
        base = wid * b_per_w

        pltpu.sync_copy(uid_hbm.at[pl.ds(rbase, n_chunks)], uidx_v)
        pltpu.sync_copy(iid_hbm.at[pl.ds(rbase, n_chunks)], iidx_v)

        for p in range(n_chunks // 2):
            copies = []
            for k in range(2):
                j = 2 * p + k
                copies.append(pltpu.async_copy(
                    qu_hbm.at[uidx_v.at[j]], urows_v.at[k], sem))
                copies.append(pltpu.async_copy(
                    qi_hbm.at[iidx_v.at[j]], irows_v.at[k], sem))
            for c in copies:
                c.wait()
            off = base + p * 2 * _IDX_CHUNK
            for k in range(2):
                pltpu.sync_copy(
                    urows_v.at[k],
                    out_u.at[pl.ds(off + k * _IDX_CHUNK, _IDX_CHUNK)])
                pltpu.sync_copy(
                    irows_v.at[k],
                    out_i.at[pl.ds(off + k * _IDX_CHUNK, _IDX_CHUNK)])

    return gather_kernel


def _unpack_half(q_ref, s_ref):
    q = q_ref[...]
    s = s_ref[...]
    ab = jnp.where(s == 0, q[:, 0:32], q[:, 32:64])
    cd = jnp.where(s == 2, q[:, 64:96], q[:, 96:128])
    bits = lax.bitcast_convert_type(
        jnp.where(s < 2, ab, cd), jnp.uint32)
    hi = lax.bitcast_convert_type(bits & jnp.uint32(0xFFFF0000), jnp.float32)
    lo = lax.bitcast_convert_type(bits << 16, jnp.float32)
    return hi, lo


def _mlp_body(uq_ref, iq_ref, su_ref, si_ref,
              w1a_ref, w1b_ref, w1c_ref, w1d_ref, b1_ref, w2_ref, b2_ref,
              w3_ref, b3_ref, o_ref):
    dot = functools.partial(jnp.dot, preferred_element_type=jnp.float32)
    uhi, ulo = _unpack_half(uq_ref, su_ref)
    ihi, ilo = _unpack_half(iq_ref, si_ref)
    h = dot(uhi, w1a_ref[...]) + dot(ulo, w1b_ref[...])
    h = h + dot(ihi, w1c_ref[...]) + dot(ilo, w1d_ref[...])
    h = jnp.maximum(h + b1_ref[...], 0.0)
    h2 = jnp.maximum(dot(h, w2_ref[...]) + b2_ref[...], 0.0)
    z = jnp.sum(h2 * w3_ref[...], axis=1) + b3_ref[0, 0]
    o_ref[...] = 1.0 / (1.0 + jnp.exp(-z))


def _mlp(uq, iq, su, si, w1a, w1b, w1c, w1d, b1, w2, b2, w3, b3):
    blk = 8192
    grid = (B // blk,)
    return pl.pallas_call(
        _mlp_body,
        grid=grid,
        in_specs=[
            pl.BlockSpec((blk, 128), lambda k: (k, 0)),
            pl.BlockSpec((blk, 128), lambda k: (k, 0)),
            pl.BlockSpec((blk, 1), lambda k: (k, 0)),
            pl.BlockSpec((blk, 1), lambda k: (k, 0)),
            pl.BlockSpec((EMB // 2, HID), lambda k: (0, 0)),
            pl.BlockSpec((EMB // 2, HID), lambda k: (0, 0)),
            pl.BlockSpec((EMB // 2, HID), lambda k: (0, 0)),
            pl.BlockSpec((EMB // 2, HID), lambda k: (0, 0)),
            pl.BlockSpec((1, HID), lambda k: (0, 0)),
            pl.BlockSpec((HID, HID // 2), lambda k: (0, 0)),
            pl.BlockSpec((1, HID // 2), lambda k: (0, 0)),
            pl.BlockSpec((1, HID // 2), lambda k: (0, 0)),
            pl.BlockSpec((1, 1), lambda k: (0, 0)),
        ],
        out_specs=pl.BlockSpec((blk,), lambda k: (k,)),
        out_shape=jax.ShapeDtypeStruct((B,), jnp.float32),
    )(uq, iq, su, si, w1a, w1b, w1c, w1d, b1, w2, b2, w3, b3)


def kernel(user_ids, item_ids, user_emb, item_emb, W1, b1, W2, b2, W3, b3):
    uq_row = (user_ids // _TW) * _QB + (user_ids % _QB)
    iq_row = (item_ids // _TW) * _QB + (item_ids % _QB)
    uidq = uq_row.reshape(B // _IDX_CHUNK, _IDX_CHUNK)
    iidq = iq_row.reshape(B // _IDX_CHUNK, _IDX_CHUNK)
    qu, qi = _transpack(user_emb.T, item_emb.T)
    uq, iq = _make_gather()(qu, qi, uidq, iidq)
    su = ((user_ids % _TW) // _QB).reshape(B, 1)
    si = ((item_ids % _TW) // _QB).reshape(B, 1)
    return _mlp(uq, iq, su, si,
                W1[0:32], W1[32:64], W1[64:96], W1[96:128],
                b1.reshape(1, HID), W2, b2.reshape(1, HID // 2),
                W3.reshape(1, HID // 2), b3.reshape(1, 1))


# speedup vs baseline: 1.0908x; 1.0172x over previous
import functools

import jax
import jax.numpy as jnp
from jax import lax
from jax.experimental import pallas as pl
from jax.experimental.pallas import tpu as pltpu
from jax.experimental.pallas import tpu_sc as plsc

B = 16384
EMB = 64
HID = 128
NV = 1000000

_IDX_CHUNK = 128

_TW = 32768
_TGRID = -(-NV // _TW)
_QB = _TW // 4
NQ = _TGRID * _QB


def _transpack_body(u_ref, i_ref, qu_ref, qi_ref):
    def rne(v):
        b = lax.bitcast_convert_type(v, jnp.uint32)
        return (b + jnp.uint32(0x7FFF) + ((b >> 16) & jnp.uint32(1))) \
            & jnp.uint32(0xFFFF0000)

    for src, dst in ((u_ref, qu_ref), (i_ref, qi_ref)):
        x = src[...]
        hi = rne(x[: EMB // 2, :])
        lo = rne(x[EMB // 2:, :])
        w = hi | (lo >> 16)
        wf = lax.bitcast_convert_type(w, jnp.float32)
        wb = jnp.concatenate(
            [wf[:, s * _QB:(s + 1) * _QB] for s in range(4)], axis=0)
        dst[...] = wb.T


def _transpack(ut, it):
    return pl.pallas_call(
        _transpack_body,
        grid=(_TGRID,),
        in_specs=[
            pl.BlockSpec((EMB, _TW), lambda j: (0, j)),
            pl.BlockSpec((EMB, _TW), lambda j: (0, j)),
        ],
        out_specs=[
            pl.BlockSpec((_QB, 128), lambda j: (j, 0)),
            pl.BlockSpec((_QB, 128), lambda j: (j, 0)),
        ],
        out_shape=[
            jax.ShapeDtypeStruct((NQ, 128), jnp.float32),
            jax.ShapeDtypeStruct((NQ, 128), jnp.float32),
        ],
    )(ut, it)


def _make_gather():
    nc, ns = 2, 16
    nw = nc * ns
    b_per_w = B // nw
    n_chunks = b_per_w // _IDX_CHUNK

    mesh = plsc.VectorSubcoreMesh(core_axis_name="c", subcore_axis_name="s")

    @functools.partial(
        pl.kernel,
        out_type=[
            jax.ShapeDtypeStruct((B, 128), jnp.float32),
            jax.ShapeDtypeStruct((B, 128), jnp.float32),
        ],
        mesh=mesh,
        scratch_types=[
            pltpu.VMEM((n_chunks, _IDX_CHUNK), jnp.int32),
            pltpu.VMEM((n_chunks, _IDX_CHUNK), jnp.int32),
            pltpu.VMEM((2, _IDX_CHUNK, 128), jnp.float32),
            pltpu.VMEM((2, _IDX_CHUNK, 128), jnp.float32),
            pltpu.SemaphoreType.DMA,
        ],
    )
    def gather_kernel(qu_hbm, qi_hbm, uid_hbm, iid_hbm, out_u, out_i,
                      uidx_v, iidx_v, urows_v, irows_v, sem):
        wid = lax.axis_index("s") * nc + lax.axis_index("c")
        rbase = wid * n_chunks
        base = wid * b_per_w

        pltpu.sync_copy(uid_hbm.at[pl.ds(rbase, n_chunks)], uidx_v)
        pltpu.sync_copy(iid_hbm.at[pl.ds(rbase, n_chunks)], iidx_v)

        for p in range(n_chunks // 2):
            copies = []
            for k in range(2):
                j = 2 * p + k
                copies.append(pltpu.async_copy(
                    qu_hbm.at[uidx_v.at[j]], urows_v.at[k], sem))
                copies.append(pltpu.async_copy(
                    qi_hbm.at[iidx_v.at[j]], irows_v.at[k], sem))
            for c in copies:
                c.wait()
            off = base + p * 2 * _IDX_CHUNK
            for k in range(2):
                pltpu.sync_copy(
                    urows_v.at[k],
                    out_u.at[pl.ds(off + k * _IDX_CHUNK, _IDX_CHUNK)])
                pltpu.sync_copy(
                    irows_v.at[k],
                    out_i.at[pl.ds(off + k * _IDX_CHUNK, _IDX_CHUNK)])

    return gather_kernel


def _unpack_half(q_ref, s):
    q = q_ref[...]
    ab = jnp.where(s == 0, q[:, 0:32], q[:, 32:64])
    cd = jnp.where(s == 2, q[:, 64:96], q[:, 96:128])
    bits = lax.bitcast_convert_type(
        jnp.where(s < 2, ab, cd), jnp.uint32)
    hi = lax.bitcast_convert_type(bits & jnp.uint32(0xFFFF0000), jnp.float32)
    lo = lax.bitcast_convert_type(bits << 16, jnp.float32)
    return hi, lo


def _mlp_body(uq_ref, iq_ref, s_ref,
              w1a_ref, w1b_ref, w1c_ref, w1d_ref, b1_ref, w2_ref, b2_ref,
              w3_ref, b3_ref, o_ref):
    dot = functools.partial(jnp.dot, preferred_element_type=jnp.float32)
    s = s_ref[...]
    uhi, ulo = _unpack_half(uq_ref, s & 3)
    ihi, ilo = _unpack_half(iq_ref, s >> 2)
    h = dot(uhi, w1a_ref[...]) + dot(ulo, w1b_ref[...])
    h = h + dot(ihi, w1c_ref[...]) + dot(ilo, w1d_ref[...])
    h = jnp.maximum(h + b1_ref[...], 0.0)
    h2 = jnp.maximum(dot(h, w2_ref[...]) + b2_ref[...], 0.0)
    z = jnp.sum(h2 * w3_ref[...], axis=1) + b3_ref[0, 0]
    o_ref[...] = 1.0 / (1.0 + jnp.exp(-z))


def _mlp(uq, iq, s, w1a, w1b, w1c, w1d, b1, w2, b2, w3, b3):
    blk = 8192
    grid = (B // blk,)
    return pl.pallas_call(
        _mlp_body,
        grid=grid,
        in_specs=[
            pl.BlockSpec((blk, 128), lambda k: (k, 0)),
            pl.BlockSpec((blk, 128), lambda k: (k, 0)),
            pl.BlockSpec((blk, 1), lambda k: (k, 0)),
            pl.BlockSpec((EMB // 2, HID), lambda k: (0, 0)),
            pl.BlockSpec((EMB // 2, HID), lambda k: (0, 0)),
            pl.BlockSpec((EMB // 2, HID), lambda k: (0, 0)),
            pl.BlockSpec((EMB // 2, HID), lambda k: (0, 0)),
            pl.BlockSpec((1, HID), lambda k: (0, 0)),
            pl.BlockSpec((HID, HID // 2), lambda k: (0, 0)),
            pl.BlockSpec((1, HID // 2), lambda k: (0, 0)),
            pl.BlockSpec((1, HID // 2), lambda k: (0, 0)),
            pl.BlockSpec((1, 1), lambda k: (0, 0)),
        ],
        out_specs=pl.BlockSpec((blk,), lambda k: (k,)),
        out_shape=jax.ShapeDtypeStruct((B,), jnp.float32),
    )(uq, iq, s, w1a, w1b, w1c, w1d, b1, w2, b2, w3, b3)


def kernel(user_ids, item_ids, user_emb, item_emb, W1, b1, W2, b2, W3, b3):
    uq_row = (user_ids // _TW) * _QB + (user_ids % _QB)
    iq_row = (item_ids // _TW) * _QB + (item_ids % _QB)
    uidq = uq_row.reshape(B // _IDX_CHUNK, _IDX_CHUNK)
    iidq = iq_row.reshape(B // _IDX_CHUNK, _IDX_CHUNK)
    qu, qi = _transpack(user_emb.T, item_emb.T)
    uq, iq = _make_gather()(qu, qi, uidq, iidq)
    su = (user_ids % _TW) // _QB
    si = (item_ids % _TW) // _QB
    s = (su | (si << 2)).reshape(B, 1)
    return _mlp(uq, iq, s,
                W1[0:32], W1[32:64], W1[64:96], W1[96:128],
                b1.reshape(1, HID), W2, b2.reshape(1, HID // 2),
                W3.reshape(1, HID // 2), b3.reshape(1, 1))
